# baseline (speedup 1.0000x reference)
import functools

import jax
import jax.numpy as jnp
from jax import lax
from jax.experimental import pallas as pl
from jax.experimental.pallas import tpu as pltpu
from jax.experimental.pallas import tpu_sc as plsc

B, S, C = 4096, 26, 1000
NW = 32
RPW = B // NW
IPW = RPW * S
NIDX = 128
NCHUNK = IPW // NIDX


def _sc_body(x_hbm, out_ref, x_vmem, idx_vmem, ones_vmem, sem):
    cid = lax.axis_index("c")
    sid = lax.axis_index("s")
    wid = sid * 2 + cid
    base = wid * RPW

    pltpu.sync_copy(x_hbm.at[pl.ds(base, RPW)], x_vmem)

    for t in range(8):
        ones_vmem[pl.ds(t * 16, 16)] = jnp.full((16,), 1.0, jnp.float32)

    lanes = lax.iota(jnp.int32, 16)

    def fill_chunk(j, carry):
        for c in range(8):
            k = j * NIDX + c * 16 + lanes
            row = k // S
            s = k - row * S
            xv = plsc.load_gather(x_vmem, [row, s])
            q = ((base + row) * S + s) * C + xv
            plsc.store_scatter(idx_vmem, [jnp.full((16,), j, jnp.int32),
                                          c * 16 + lanes], q)
        return carry

    lax.fori_loop(0, NCHUNK, fill_chunk, 0)

    for j in range(NCHUNK):
        pltpu.async_copy(ones_vmem, out_ref.at[idx_vmem.at[j]], sem)
    for j in range(NCHUNK):
        pltpu.make_async_copy(ones_vmem, out_ref.at[idx_vmem.at[j]], sem).wait()


def kernel(x):
    x2 = x.astype(jnp.int32)
    mesh = plsc.VectorSubcoreMesh(core_axis_name="c", subcore_axis_name="s")
    run = functools.partial(
        pl.kernel,
        mesh=mesh,
        out_type=(),
        scratch_types=[
            pltpu.VMEM((RPW, S), jnp.int32),
            pltpu.VMEM((NCHUNK, NIDX), jnp.int32),
            pltpu.VMEM((NIDX,), jnp.float32),
            pltpu.SemaphoreType.DMA,
        ],
        compiler_params=pltpu.CompilerParams(needs_layout_passes=False),
    )(_sc_body)

    out_ref = jax.new_ref(jnp.zeros((B * S * C,), jnp.float32))
    run(x2, out_ref)
    return out_ref[...].reshape(B, S, C)


# speedup vs baseline: 2.0253x; 2.0253x over previous
import functools

import jax
import jax.numpy as jnp
from jax import lax
from jax.experimental import pallas as pl
from jax.experimental.pallas import tpu as pltpu

N_CLASSES = 1000
NBUF = 4
RB = 64


def _rcopy(buf, slot, o_hbm, dst, sems):
    return pltpu.make_async_remote_copy(
        buf.at[slot],
        o_hbm.at[pl.ds(dst * RB, RB)],
        sems.at[slot, 0],
        sems.at[slot, 1],
        device_id=0,
        device_id_type=pltpu.DeviceIdType.LOGICAL,
    )


def _onehot_body(x_ref, o_hbm, buf, sems):
    i = pl.program_id(0)
    g = pl.num_programs(0)
    slot = lax.rem(i, NBUF)

    @pl.when(i >= NBUF)
    def _drain():
        c = _rcopy(buf, slot, o_hbm, slot, sems)
        c.wait_send()
        c.wait_recv()

    idx = x_ref[...]
    classes = lax.broadcasted_iota(jnp.int32, (RB, 26, N_CLASSES), 2)
    buf[slot] = (classes == idx).astype(jnp.float32)

    _rcopy(buf, slot, o_hbm, i, sems).start()

    @pl.when(i == g - 1)
    def _final():
        for s in range(NBUF):
            c = _rcopy(buf, s, o_hbm, s, sems)
            c.wait_send()
            c.wait_recv()


def kernel(x):
    B, S = x.shape
    x3 = x.reshape(B, S, 1).astype(jnp.int32)
    out = pl.pallas_call(
        _onehot_body,
        grid=(B // RB,),
        in_specs=[pl.BlockSpec((RB, S, 1), lambda i: (i, 0, 0))],
        out_specs=pl.BlockSpec(memory_space=pl.ANY),
        out_shape=jax.ShapeDtypeStruct((B, S, N_CLASSES), jnp.float32),
        scratch_shapes=[
            pltpu.VMEM((NBUF, RB, S, N_CLASSES), jnp.float32),
            pltpu.SemaphoreType.DMA((NBUF, 2)),
        ],
    )(x3)
    return out


# speedup vs baseline: 2.0960x; 1.0349x over previous
import functools

import jax
import jax.numpy as jnp
from jax import lax
from jax.experimental import pallas as pl
from jax.experimental.pallas import tpu as pltpu
from jax.experimental.pallas import tpu_sc as plsc

B, S, C = 4096, 26, 1000
NW = 32
RPW = B // NW
CB = 1
NB = 2
NCH = RPW // CB


def _scatter_chunk(buf, x_vmem, c, value):
    lanes0 = lax.iota(jnp.int32, 16)
    lanes1 = lanes0 + 16
    mask0 = lanes0 < S
    mask1 = lanes1 < S
    val = jnp.full((16,), value, jnp.float32)
    for r in range(CB):
        row = c * CB + r
        rvec_buf = jnp.full((16,), r, jnp.int32)
        rvec_x = jnp.full((16,), row, jnp.int32)
        for lanes, mask in ((lanes0, mask0), (lanes1, mask1)):
            xv = plsc.load_gather(x_vmem, [rvec_x, lanes], mask=mask)
            plsc.store_scatter(buf, [rvec_buf, lanes, xv], val, mask=mask)


def _sc_body(x_hbm, z_hbm, out_hbm, x_vmem, buf, sems):
    cid = lax.axis_index("c")
    sid = lax.axis_index("s")
    wid = sid * 2 + cid
    base = wid * RPW

    pltpu.sync_copy(x_hbm.at[pl.ds(base, RPW)], x_vmem)
    for b in range(NB):
        pltpu.sync_copy(z_hbm, buf.at[b])

    def step(g, carry):
        for b in range(NB):
            c = g * NB + b

            @pl.when(g > 0)
            def _reuse():
                pltpu.make_async_copy(
                    buf.at[b], out_hbm.at[pl.ds(base, CB)], sems.at[b]
                ).wait()
                _scatter_chunk(buf.at[b], x_vmem, c - NB, 0.0)

            _scatter_chunk(buf.at[b], x_vmem, c, 1.0)
            pltpu.async_copy(
                buf.at[b], out_hbm.at[pl.ds(base + c * CB, CB)], sems.at[b]
            )
        return carry

    lax.fori_loop(0, NCH // NB, step, 0)
    for b in range(NB):
        pltpu.make_async_copy(
            buf.at[b], out_hbm.at[pl.ds(base, CB)], sems.at[b]
        ).wait()


def kernel(x):
    z = jnp.zeros((CB, S, C), jnp.float32)
    mesh = plsc.VectorSubcoreMesh(core_axis_name="c", subcore_axis_name="s")
    run = functools.partial(
        pl.kernel,
        mesh=mesh,
        out_type=jax.ShapeDtypeStruct((B, S, C), jnp.float32),
        scratch_types=[
            pltpu.VMEM((RPW, S), jnp.int32),
            pltpu.VMEM((NB, CB, S, C), jnp.float32),
            pltpu.SemaphoreType.DMA((NB,)),
        ],
        compiler_params=pltpu.CompilerParams(needs_layout_passes=False),
    )(_sc_body)
    return run(x.astype(jnp.int32), z)
